# baseline (speedup 1.0000x reference)
import jax
import jax.numpy as jnp
from jax import lax
from jax.experimental import pallas as pl
from jax.experimental.pallas import tpu as pltpu
from jax.experimental.pallas import tpu_sc as plsc

_N = 10000
_NPAD = 10240
_E = 160000
_FC = 128
_NPLANE = 4
_EB = 80
_EROWS = 2048
_NB = _EROWS // 16
_NSEG = 8
_SB = _NB // _NSEG
_RPT = _NPAD // 16

_BM = 1000


def _seg_body(ps, src2, dst2, ew2, outs,
              srck, dstt, ewt, r0b, r1b, r2b, r3b,
              sg0, sg1, sg2, sg3, ss0, ss1, ss2, ss3, acc):
    c = lax.axis_index("c")
    s = lax.axis_index("s")
    rows = (r0b, r1b, r2b, r3b)
    sg = (sg0, sg1, sg2, sg3)
    ss = (ss0, ss1, ss2, ss3)

    rb = s * _NB
    row0 = s * _RPT
    zero16 = jnp.zeros((16,), jnp.float32)

    def chunk_body(kk, cc):
        k = c * 2 + kk
        koff = (k * _NPAD).astype(jnp.int32)

        def zr(i, c2):
            for j8 in range(8):
                r0b[i, pl.ds(j8 * 16, 16)] = zero16
            return c2

        lax.fori_loop(0, _EB, zr, 0)
        for m in range(_RPT // _EB):
            pltpu.sync_copy(r0b, acc.at[pl.ds(row0 + m * _EB, _EB)])
        plsc.subcore_barrier()

        def g_start(j, b):
            pltpu.async_copy(ps.at[srck.at[j]], rows[b], sg[b])

        def g_wait(b):
            pltpu.make_async_copy(ps.at[pl.ds(0, _EB)], rows[b],
                                  sg[b]).wait()

        def s_start(j, b):
            pltpu.async_copy(rows[b], acc.at[dstt.at[j]], ss[b], add=True)

        def s_wait(b):
            pltpu.make_async_copy(ps.at[pl.ds(0, _EB)], rows[b],
                                  ss[b]).wait()

        for seg in range(_NSEG):
            sb = rb + seg * _SB
            pltpu.async_copy(src2.at[pl.ds(sb, _SB)], srck, sg[0])
            pltpu.async_copy(dst2.at[pl.ds(sb, _SB)], dstt, sg[1])
            pltpu.async_copy(ew2.at[pl.ds(sb, _SB)], ewt, sg[2])
            pltpu.make_async_copy(src2.at[pl.ds(sb, _SB)], srck,
                                  sg[0]).wait()
            pltpu.make_async_copy(dst2.at[pl.ds(sb, _SB)], dstt,
                                  sg[1]).wait()
            pltpu.make_async_copy(ew2.at[pl.ds(sb, _SB)], ewt,
                                  sg[2]).wait()

            kv = jnp.full((16,), 0, jnp.int32) + koff

            def addk(i, c2):
                for g in range(_EB // 16):
                    srck[i, pl.ds(g * 16, 16)] = (
                        srck[i, pl.ds(g * 16, 16)] + kv)
                return c2

            lax.fori_loop(0, _SB, addk, 0)

            g_start(0, 0)
            g_start(1, 1)
            g_start(2, 2)

            def ring_iter(it, c2):
                for b in range(4):
                    j = it * 4 + b
                    nb = (b + 3) % 4

                    @pl.when((j >= 1) & (j + 3 < _SB))
                    def _sw():
                        s_wait(nb)

                    @pl.when(j + 3 < _SB)
                    def _gs():
                        g_start(j + 3, nb)

                    g_wait(b)

                    def grp(g, c3):
                        base = pl.multiple_of(g * 16, 16)
                        wvec = ewt[j, pl.ds(base, 16)]
                        buf = rows[b]
                        for i16 in range(16):
                            wv = jnp.broadcast_to(wvec[i16:i16 + 1], (16,))
                            r = base + i16
                            for j8 in range(8):
                                buf[r, pl.ds(j8 * 16, 16)] = (
                                    buf[r, pl.ds(j8 * 16, 16)] * wv)
                        return c3

                    lax.fori_loop(0, _EB // 16, grp, 0)
                    s_start(j, b)
                return c2

            lax.fori_loop(0, _SB // 4, ring_iter, 0)
            for b in range(4):
                s_wait(b)

        plsc.subcore_barrier()
        for m in range(_RPT // _EB):
            r0 = row0 + m * _EB
            pltpu.sync_copy(acc.at[pl.ds(r0, _EB)], r0b)
            pltpu.sync_copy(r0b, outs.at[pl.ds(koff + r0, _EB)])
        return cc

    lax.fori_loop(0, _NPLANE // 2, chunk_body, 0)


_seg = pl.kernel(
    _seg_body,
    out_type=jax.ShapeDtypeStruct((_NPLANE * _NPAD, _FC), jnp.float32),
    mesh=plsc.VectorSubcoreMesh(core_axis_name="c", subcore_axis_name="s"),
    scratch_types=[
        pltpu.VMEM((_SB, _EB), jnp.int32),
        pltpu.VMEM((_SB, _EB), jnp.int32),
        pltpu.VMEM((_SB, _EB), jnp.float32),
        pltpu.VMEM((_EB, _FC), jnp.float32),
        pltpu.VMEM((_EB, _FC), jnp.float32),
        pltpu.VMEM((_EB, _FC), jnp.float32),
        pltpu.VMEM((_EB, _FC), jnp.float32),
        pltpu.SemaphoreType.DMA,
        pltpu.SemaphoreType.DMA,
        pltpu.SemaphoreType.DMA,
        pltpu.SemaphoreType.DMA,
        pltpu.SemaphoreType.DMA,
        pltpu.SemaphoreType.DMA,
        pltpu.SemaphoreType.DMA,
        pltpu.SemaphoreType.DMA,
        pltpu.VMEM_SHARED((_NPAD, _FC), jnp.float32),
    ],
)


def _leaky(x):
    return jnp.where(x >= 0, x, 0.01 * x)


def _mm_in_body(x_ref, w_ref, o_ref):
    y = jnp.dot(x_ref[...], w_ref[...], preferred_element_type=jnp.float32)
    for kk in range(_NPLANE):
        o_ref[kk] = y[:, kk * _FC:(kk + 1) * _FC]


def _mm_mid_body(a_ref, b_ref, w_ref, o_ref):
    h = jnp.concatenate([a_ref[kk] for kk in range(_NPLANE)], axis=1)
    h = _leaky(h + b_ref[...])
    y = jnp.dot(h, w_ref[...], preferred_element_type=jnp.float32)
    for kk in range(_NPLANE):
        o_ref[kk] = y[:, kk * _FC:(kk + 1) * _FC]


def _mm_out_body(a_ref, b_ref, w_ref, bl_ref, o_ref):
    h = jnp.concatenate([a_ref[kk] for kk in range(_NPLANE)], axis=1)
    h = _leaky(h + b_ref[...])
    o_ref[...] = (jnp.dot(h, w_ref[...], preferred_element_type=jnp.float32)
                  + bl_ref[...])


_STACK_SPEC = pl.BlockSpec((_NPLANE, _BM, _FC), lambda i: (0, i, 0))


def _mm_in(x, w):
    kin = x.shape[1]
    return pl.pallas_call(
        _mm_in_body,
        grid=(_N // _BM,),
        in_specs=[
            pl.BlockSpec((_BM, kin), lambda i: (i, 0)),
            pl.BlockSpec((kin, w.shape[1]), lambda i: (0, 0)),
        ],
        out_specs=_STACK_SPEC,
        out_shape=jax.ShapeDtypeStruct((_NPLANE, _NPAD, _FC), jnp.float32),
    )(x, w)


def _mm_mid(a, b, w):
    return pl.pallas_call(
        _mm_mid_body,
        grid=(_N // _BM,),
        in_specs=[
            _STACK_SPEC,
            pl.BlockSpec((1, b.shape[1]), lambda i: (0, 0)),
            pl.BlockSpec((w.shape[0], w.shape[1]), lambda i: (0, 0)),
        ],
        out_specs=_STACK_SPEC,
        out_shape=jax.ShapeDtypeStruct((_NPLANE, _NPAD, _FC), jnp.float32),
    )(a, b, w)


def _mm_out(a, b, w, bl):
    ncls = w.shape[1]
    return pl.pallas_call(
        _mm_out_body,
        grid=(_N // _BM,),
        in_specs=[
            _STACK_SPEC,
            pl.BlockSpec((1, b.shape[1]), lambda i: (0, 0)),
            pl.BlockSpec((w.shape[0], ncls), lambda i: (0, 0)),
            pl.BlockSpec((1, ncls), lambda i: (0, 0)),
        ],
        out_specs=pl.BlockSpec((_BM, ncls), lambda i: (i, 0)),
        out_shape=jax.ShapeDtypeStruct((_N, ncls), jnp.float32),
    )(a, b, w, bl)


def kernel(x, edge_index, aw0, aw1, W1, b1, W2, b2, Wl, bl):
    npad = _EROWS * _EB - _E
    src2 = jnp.pad(edge_index[0], (0, npad)).reshape(_EROWS, _EB)
    dst2 = jnp.pad(edge_index[1], (0, npad)).reshape(_EROWS, _EB)
    ew0 = jnp.pad(aw0, (0, npad)).reshape(_EROWS, _EB)
    ew1 = jnp.pad(aw1, (0, npad)).reshape(_EROWS, _EB)
    flat = (_NPLANE * _NPAD, _FC)
    stck = (_NPLANE, _NPAD, _FC)
    p = _mm_in(x, W1)
    a = _seg(p.reshape(flat), src2, dst2, ew0)
    q = _mm_mid(a.reshape(stck), b1.reshape(1, -1), W2)
    g = _seg(q.reshape(flat), src2, dst2, ew1)
    return _mm_out(g.reshape(stck), b2.reshape(1, -1), Wl, bl.reshape(1, -1))


# speedup vs baseline: 1.0207x; 1.0207x over previous
import jax
import jax.numpy as jnp
from jax import lax
from jax.experimental import pallas as pl
from jax.experimental.pallas import tpu as pltpu
from jax.experimental.pallas import tpu_sc as plsc

_N = 10000
_NPAD = 10240
_E = 160000
_FC = 128
_NPLANE = 4
_EB = 80
_EROWS = 2048
_NB = _EROWS // 16
_NSEG = 8
_SB = _NB // _NSEG
_RPT = _NPAD // 16

_BM = 1000


def _seg_body(ps, src2, dst2, ew2, outs,
              srck, dstt, ewt, r0b, r1b, r2b, r3b,
              sg0, sg1, sg2, sg3, ss0, ss1, ss2, ss3, acc):
    c = lax.axis_index("c")
    s = lax.axis_index("s")
    rows = (r0b, r1b, r2b, r3b)
    sg = (sg0, sg1, sg2, sg3)
    ss = (ss0, ss1, ss2, ss3)

    rb = s * _NB
    row0 = s * _RPT
    zero16 = jnp.zeros((16,), jnp.float32)

    def chunk_body(kk, cc):
        k = c * 2 + kk
        koff = (k * _NPAD).astype(jnp.int32)

        def zr(i, c2):
            for j8 in range(8):
                r0b[i, pl.ds(j8 * 16, 16)] = zero16
            return c2

        lax.fori_loop(0, _EB, zr, 0)
        for m in range(_RPT // _EB):
            pltpu.sync_copy(r0b, acc.at[pl.ds(row0 + m * _EB, _EB)])
        plsc.subcore_barrier()

        def g_start(j, b):
            pltpu.async_copy(ps.at[srck.at[j]], rows[b], sg[b])

        def g_wait(b):
            pltpu.make_async_copy(ps.at[pl.ds(0, _EB)], rows[b],
                                  sg[b]).wait()

        def s_start(j, b):
            pltpu.async_copy(rows[b], acc.at[dstt.at[j]], ss[b], add=True)

        def s_wait(b):
            pltpu.make_async_copy(ps.at[pl.ds(0, _EB)], rows[b],
                                  ss[b]).wait()

        for seg in range(_NSEG):
            sb = rb + seg * _SB
            pltpu.async_copy(src2.at[pl.ds(sb, _SB)], srck, sg[0])
            pltpu.async_copy(dst2.at[pl.ds(sb, _SB)], dstt, sg[1])
            pltpu.async_copy(ew2.at[pl.ds(sb, _SB)], ewt, sg[2])
            pltpu.make_async_copy(src2.at[pl.ds(sb, _SB)], srck,
                                  sg[0]).wait()
            pltpu.make_async_copy(dst2.at[pl.ds(sb, _SB)], dstt,
                                  sg[1]).wait()
            pltpu.make_async_copy(ew2.at[pl.ds(sb, _SB)], ewt,
                                  sg[2]).wait()

            kv = jnp.full((16,), 0, jnp.int32) + koff

            def addk(i, c2):
                for g in range(_EB // 16):
                    srck[i, pl.ds(g * 16, 16)] = (
                        srck[i, pl.ds(g * 16, 16)] + kv)
                return c2

            lax.fori_loop(0, _SB, addk, 0)

            g_start(0, 0)
            g_start(1, 1)

            def ring_iter(it, c2):
                for b in range(4):
                    j = it * 4 + b
                    nb = (b + 2) % 4

                    @pl.when((j >= 2) & (j + 2 < _SB))
                    def _sw():
                        s_wait(nb)

                    @pl.when(j + 2 < _SB)
                    def _gs():
                        g_start(j + 2, nb)

                    g_wait(b)

                    def grp(g, c3):
                        base = pl.multiple_of(g * 16, 16)
                        wvec = ewt[j, pl.ds(base, 16)]
                        buf = rows[b]
                        for i16 in range(16):
                            wv = jnp.broadcast_to(wvec[i16:i16 + 1], (16,))
                            r = base + i16
                            for j8 in range(8):
                                buf[r, pl.ds(j8 * 16, 16)] = (
                                    buf[r, pl.ds(j8 * 16, 16)] * wv)
                        return c3

                    lax.fori_loop(0, _EB // 16, grp, 0)
                    s_start(j, b)
                return c2

            lax.fori_loop(0, _SB // 4, ring_iter, 0)
            for b in range(4):
                s_wait(b)

        plsc.subcore_barrier()
        for m in range(_RPT // _EB):
            r0 = row0 + m * _EB
            pltpu.sync_copy(acc.at[pl.ds(r0, _EB)], r0b)
            pltpu.sync_copy(r0b, outs.at[pl.ds(koff + r0, _EB)])
        return cc

    lax.fori_loop(0, _NPLANE // 2, chunk_body, 0)


_seg = pl.kernel(
    _seg_body,
    out_type=jax.ShapeDtypeStruct((_NPLANE * _NPAD, _FC), jnp.float32),
    mesh=plsc.VectorSubcoreMesh(core_axis_name="c", subcore_axis_name="s"),
    scratch_types=[
        pltpu.VMEM((_SB, _EB), jnp.int32),
        pltpu.VMEM((_SB, _EB), jnp.int32),
        pltpu.VMEM((_SB, _EB), jnp.float32),
        pltpu.VMEM((_EB, _FC), jnp.float32),
        pltpu.VMEM((_EB, _FC), jnp.float32),
        pltpu.VMEM((_EB, _FC), jnp.float32),
        pltpu.VMEM((_EB, _FC), jnp.float32),
        pltpu.SemaphoreType.DMA,
        pltpu.SemaphoreType.DMA,
        pltpu.SemaphoreType.DMA,
        pltpu.SemaphoreType.DMA,
        pltpu.SemaphoreType.DMA,
        pltpu.SemaphoreType.DMA,
        pltpu.SemaphoreType.DMA,
        pltpu.SemaphoreType.DMA,
        pltpu.VMEM_SHARED((_NPAD, _FC), jnp.float32),
    ],
)


def _leaky(x):
    return jnp.where(x >= 0, x, 0.01 * x)


def _mm_in_body(x_ref, w_ref, o_ref):
    y = jnp.dot(x_ref[...], w_ref[...], preferred_element_type=jnp.float32)
    for kk in range(_NPLANE):
        o_ref[kk] = y[:, kk * _FC:(kk + 1) * _FC]


def _mm_mid_body(a_ref, b_ref, w_ref, o_ref):
    h = jnp.concatenate([a_ref[kk] for kk in range(_NPLANE)], axis=1)
    h = _leaky(h + b_ref[...])
    y = jnp.dot(h, w_ref[...], preferred_element_type=jnp.float32)
    for kk in range(_NPLANE):
        o_ref[kk] = y[:, kk * _FC:(kk + 1) * _FC]


def _mm_out_body(a_ref, b_ref, w_ref, bl_ref, o_ref):
    h = jnp.concatenate([a_ref[kk] for kk in range(_NPLANE)], axis=1)
    h = _leaky(h + b_ref[...])
    o_ref[...] = (jnp.dot(h, w_ref[...], preferred_element_type=jnp.float32)
                  + bl_ref[...])


_STACK_SPEC = pl.BlockSpec((_NPLANE, _BM, _FC), lambda i: (0, i, 0))


def _mm_in(x, w):
    kin = x.shape[1]
    return pl.pallas_call(
        _mm_in_body,
        grid=(_N // _BM,),
        in_specs=[
            pl.BlockSpec((_BM, kin), lambda i: (i, 0)),
            pl.BlockSpec((kin, w.shape[1]), lambda i: (0, 0)),
        ],
        out_specs=_STACK_SPEC,
        out_shape=jax.ShapeDtypeStruct((_NPLANE, _NPAD, _FC), jnp.float32),
    )(x, w)


def _mm_mid(a, b, w):
    return pl.pallas_call(
        _mm_mid_body,
        grid=(_N // _BM,),
        in_specs=[
            _STACK_SPEC,
            pl.BlockSpec((1, b.shape[1]), lambda i: (0, 0)),
            pl.BlockSpec((w.shape[0], w.shape[1]), lambda i: (0, 0)),
        ],
        out_specs=_STACK_SPEC,
        out_shape=jax.ShapeDtypeStruct((_NPLANE, _NPAD, _FC), jnp.float32),
    )(a, b, w)


def _mm_out(a, b, w, bl):
    ncls = w.shape[1]
    return pl.pallas_call(
        _mm_out_body,
        grid=(_N // _BM,),
        in_specs=[
            _STACK_SPEC,
            pl.BlockSpec((1, b.shape[1]), lambda i: (0, 0)),
            pl.BlockSpec((w.shape[0], ncls), lambda i: (0, 0)),
            pl.BlockSpec((1, ncls), lambda i: (0, 0)),
        ],
        out_specs=pl.BlockSpec((_BM, ncls), lambda i: (i, 0)),
        out_shape=jax.ShapeDtypeStruct((_N, ncls), jnp.float32),
    )(a, b, w, bl)


def kernel(x, edge_index, aw0, aw1, W1, b1, W2, b2, Wl, bl):
    npad = _EROWS * _EB - _E
    src2 = jnp.pad(edge_index[0], (0, npad)).reshape(_EROWS, _EB)
    dst2 = jnp.pad(edge_index[1], (0, npad)).reshape(_EROWS, _EB)
    ew0 = jnp.pad(aw0, (0, npad)).reshape(_EROWS, _EB)
    ew1 = jnp.pad(aw1, (0, npad)).reshape(_EROWS, _EB)
    flat = (_NPLANE * _NPAD, _FC)
    stck = (_NPLANE, _NPAD, _FC)
    p = _mm_in(x, W1)
    a = _seg(p.reshape(flat), src2, dst2, ew0)
    q = _mm_mid(a.reshape(stck), b1.reshape(1, -1), W2)
    g = _seg(q.reshape(flat), src2, dst2, ew1)
    return _mm_out(g.reshape(stck), b2.reshape(1, -1), Wl, bl.reshape(1, -1))


# speedup vs baseline: 1.0282x; 1.0073x over previous
import jax
import jax.numpy as jnp
from jax import lax
from jax.experimental import pallas as pl
from jax.experimental.pallas import tpu as pltpu
from jax.experimental.pallas import tpu_sc as plsc

_N = 10000
_NPAD = 10240
_E = 160000
_FC = 128
_NPLANE = 4
_EB = 80
_EROWS = 2048
_NB = _EROWS // 16
_NSEG = 8
_SB = _NB // _NSEG
_RPT = _NPAD // 16

_BM = 1000


def _seg_body(ps, src2, dst2, ew2, outs,
              srck, dstt, ewt, r0b, r1b, r2b, r3b,
              sg0, sg1, sg2, sg3, ss0, ss1, ss2, ss3, acc):
    c = lax.axis_index("c")
    s = lax.axis_index("s")
    rows = (r0b, r1b, r2b, r3b)
    sg = (sg0, sg1, sg2, sg3)
    ss = (ss0, ss1, ss2, ss3)

    rb = s * _NB
    row0 = s * _RPT
    zero16 = jnp.zeros((16,), jnp.float32)

    def chunk_body(kk, cc):
        k = c * 2 + kk
        koff = (k * _NPAD).astype(jnp.int32)

        def zr(i, c2):
            for j8 in range(8):
                r0b[i, pl.ds(j8 * 16, 16)] = zero16
            return c2

        lax.fori_loop(0, _EB, zr, 0)
        for m in range(_RPT // _EB):
            pltpu.async_copy(r0b, acc.at[pl.ds(row0 + m * _EB, _EB)],
                             sg[m % 4])
        for m in range(_RPT // _EB):
            pltpu.make_async_copy(r0b, acc.at[pl.ds(row0, _EB)],
                                  sg[m % 4]).wait()
        plsc.subcore_barrier()

        def g_start(j, b):
            pltpu.async_copy(ps.at[srck.at[j]], rows[b], sg[b])

        def g_wait(b):
            pltpu.make_async_copy(ps.at[pl.ds(0, _EB)], rows[b],
                                  sg[b]).wait()

        def s_start(j, b):
            pltpu.async_copy(rows[b], acc.at[dstt.at[j]], ss[b], add=True)

        def s_wait(b):
            pltpu.make_async_copy(ps.at[pl.ds(0, _EB)], rows[b],
                                  ss[b]).wait()

        for seg in range(_NSEG):
            sb = rb + seg * _SB
            pltpu.async_copy(src2.at[pl.ds(sb, _SB)], srck, sg[0])
            pltpu.async_copy(dst2.at[pl.ds(sb, _SB)], dstt, sg[1])
            pltpu.async_copy(ew2.at[pl.ds(sb, _SB)], ewt, sg[2])
            pltpu.make_async_copy(src2.at[pl.ds(sb, _SB)], srck,
                                  sg[0]).wait()
            pltpu.make_async_copy(dst2.at[pl.ds(sb, _SB)], dstt,
                                  sg[1]).wait()
            pltpu.make_async_copy(ew2.at[pl.ds(sb, _SB)], ewt,
                                  sg[2]).wait()

            kv = jnp.full((16,), 0, jnp.int32) + koff

            def addk(i, c2):
                for g in range(_EB // 16):
                    srck[i, pl.ds(g * 16, 16)] = (
                        srck[i, pl.ds(g * 16, 16)] + kv)
                return c2

            lax.fori_loop(0, _SB, addk, 0)

            g_start(0, 0)
            g_start(1, 1)

            def ring_iter(it, c2):
                for b in range(4):
                    j = it * 4 + b
                    nb = (b + 2) % 4

                    @pl.when((j >= 2) & (j + 2 < _SB))
                    def _sw():
                        s_wait(nb)

                    @pl.when(j + 2 < _SB)
                    def _gs():
                        g_start(j + 2, nb)

                    g_wait(b)

                    def grp(g, c3):
                        base = pl.multiple_of(g * 16, 16)
                        wvec = ewt[j, pl.ds(base, 16)]
                        buf = rows[b]
                        for i16 in range(16):
                            wv = jnp.broadcast_to(wvec[i16:i16 + 1], (16,))
                            r = base + i16
                            for j8 in range(8):
                                buf[r, pl.ds(j8 * 16, 16)] = (
                                    buf[r, pl.ds(j8 * 16, 16)] * wv)
                        return c3

                    lax.fori_loop(0, _EB // 16, grp, 0)
                    s_start(j, b)
                return c2

            lax.fori_loop(0, _SB // 4, ring_iter, 0)
            for b in range(4):
                s_wait(b)

        plsc.subcore_barrier()
        wb = (r0b, r1b)
        nwb = _RPT // _EB
        pltpu.async_copy(acc.at[pl.ds(row0, _EB)], r0b, sg[0])
        for m in range(nwb):
            bm = wb[m % 2]
            pltpu.make_async_copy(acc.at[pl.ds(row0, _EB)], bm,
                                  sg[m % 2]).wait()
            if m >= 1:
                pltpu.make_async_copy(wb[(m - 1) % 2],
                                      outs.at[pl.ds(koff + row0, _EB)],
                                      ss[(m - 1) % 2]).wait()
            if m + 1 < nwb:
                pltpu.async_copy(acc.at[pl.ds(row0 + (m + 1) * _EB, _EB)],
                                 wb[(m + 1) % 2], sg[(m + 1) % 2])
            pltpu.async_copy(bm, outs.at[pl.ds(koff + row0 + m * _EB, _EB)],
                             ss[m % 2])
        pltpu.make_async_copy(wb[(nwb - 1) % 2],
                              outs.at[pl.ds(koff + row0, _EB)],
                              ss[(nwb - 1) % 2]).wait()
        return cc

    lax.fori_loop(0, _NPLANE // 2, chunk_body, 0)


_seg = pl.kernel(
    _seg_body,
    out_type=jax.ShapeDtypeStruct((_NPLANE * _NPAD, _FC), jnp.float32),
    mesh=plsc.VectorSubcoreMesh(core_axis_name="c", subcore_axis_name="s"),
    scratch_types=[
        pltpu.VMEM((_SB, _EB), jnp.int32),
        pltpu.VMEM((_SB, _EB), jnp.int32),
        pltpu.VMEM((_SB, _EB), jnp.float32),
        pltpu.VMEM((_EB, _FC), jnp.float32),
        pltpu.VMEM((_EB, _FC), jnp.float32),
        pltpu.VMEM((_EB, _FC), jnp.float32),
        pltpu.VMEM((_EB, _FC), jnp.float32),
        pltpu.SemaphoreType.DMA,
        pltpu.SemaphoreType.DMA,
        pltpu.SemaphoreType.DMA,
        pltpu.SemaphoreType.DMA,
        pltpu.SemaphoreType.DMA,
        pltpu.SemaphoreType.DMA,
        pltpu.SemaphoreType.DMA,
        pltpu.SemaphoreType.DMA,
        pltpu.VMEM_SHARED((_NPAD, _FC), jnp.float32),
    ],
)


def _leaky(x):
    return jnp.where(x >= 0, x, 0.01 * x)


def _mm_in_body(x_ref, w_ref, o_ref):
    y = jnp.dot(x_ref[...], w_ref[...], preferred_element_type=jnp.float32)
    for kk in range(_NPLANE):
        o_ref[kk] = y[:, kk * _FC:(kk + 1) * _FC]


def _mm_mid_body(a_ref, b_ref, w_ref, o_ref):
    h = jnp.concatenate([a_ref[kk] for kk in range(_NPLANE)], axis=1)
    h = _leaky(h + b_ref[...])
    y = jnp.dot(h, w_ref[...], preferred_element_type=jnp.float32)
    for kk in range(_NPLANE):
        o_ref[kk] = y[:, kk * _FC:(kk + 1) * _FC]


def _mm_out_body(a_ref, b_ref, w_ref, bl_ref, o_ref):
    h = jnp.concatenate([a_ref[kk] for kk in range(_NPLANE)], axis=1)
    h = _leaky(h + b_ref[...])
    o_ref[...] = (jnp.dot(h, w_ref[...], preferred_element_type=jnp.float32)
                  + bl_ref[...])


_STACK_SPEC = pl.BlockSpec((_NPLANE, _BM, _FC), lambda i: (0, i, 0))


def _mm_in(x, w):
    kin = x.shape[1]
    return pl.pallas_call(
        _mm_in_body,
        grid=(_N // _BM,),
        in_specs=[
            pl.BlockSpec((_BM, kin), lambda i: (i, 0)),
            pl.BlockSpec((kin, w.shape[1]), lambda i: (0, 0)),
        ],
        out_specs=_STACK_SPEC,
        out_shape=jax.ShapeDtypeStruct((_NPLANE, _NPAD, _FC), jnp.float32),
    )(x, w)


def _mm_mid(a, b, w):
    return pl.pallas_call(
        _mm_mid_body,
        grid=(_N // _BM,),
        in_specs=[
            _STACK_SPEC,
            pl.BlockSpec((1, b.shape[1]), lambda i: (0, 0)),
            pl.BlockSpec((w.shape[0], w.shape[1]), lambda i: (0, 0)),
        ],
        out_specs=_STACK_SPEC,
        out_shape=jax.ShapeDtypeStruct((_NPLANE, _NPAD, _FC), jnp.float32),
    )(a, b, w)


def _mm_out(a, b, w, bl):
    ncls = w.shape[1]
    return pl.pallas_call(
        _mm_out_body,
        grid=(_N // _BM,),
        in_specs=[
            _STACK_SPEC,
            pl.BlockSpec((1, b.shape[1]), lambda i: (0, 0)),
            pl.BlockSpec((w.shape[0], ncls), lambda i: (0, 0)),
            pl.BlockSpec((1, ncls), lambda i: (0, 0)),
        ],
        out_specs=pl.BlockSpec((_BM, ncls), lambda i: (i, 0)),
        out_shape=jax.ShapeDtypeStruct((_N, ncls), jnp.float32),
    )(a, b, w, bl)


def kernel(x, edge_index, aw0, aw1, W1, b1, W2, b2, Wl, bl):
    npad = _EROWS * _EB - _E
    src2 = jnp.pad(edge_index[0], (0, npad)).reshape(_EROWS, _EB)
    dst2 = jnp.pad(edge_index[1], (0, npad)).reshape(_EROWS, _EB)
    ew0 = jnp.pad(aw0, (0, npad)).reshape(_EROWS, _EB)
    ew1 = jnp.pad(aw1, (0, npad)).reshape(_EROWS, _EB)
    flat = (_NPLANE * _NPAD, _FC)
    stck = (_NPLANE, _NPAD, _FC)
    p = _mm_in(x, W1)
    a = _seg(p.reshape(flat), src2, dst2, ew0)
    q = _mm_mid(a.reshape(stck), b1.reshape(1, -1), W2)
    g = _seg(q.reshape(flat), src2, dst2, ew1)
    return _mm_out(g.reshape(stck), b2.reshape(1, -1), Wl, bl.reshape(1, -1))


# speedup vs baseline: 1.0314x; 1.0031x over previous
import jax
import jax.numpy as jnp
from jax import lax
from jax.experimental import pallas as pl
from jax.experimental.pallas import tpu as pltpu
from jax.experimental.pallas import tpu_sc as plsc

_N = 10000
_NPAD = 10240
_E = 160000
_FC = 128
_NPLANE = 4
_EB = 80
_EROWS = 2048
_NB = _EROWS // 16
_NSEG = 8
_SB = _NB // _NSEG
_RPT = _NPAD // 16

_BM = 1000


def _seg_body(ps, src2, dst2, ew2, outs,
              srck, dstt, ewt, r0b, r1b, r2b, r3b,
              sg0, sg1, sg2, sg3, ss0, ss1, ss2, ss3, acc):
    c = lax.axis_index("c")
    s = lax.axis_index("s")
    rows = (r0b, r1b, r2b, r3b)
    sg = (sg0, sg1, sg2, sg3)
    ss = (ss0, ss1, ss2, ss3)

    rb = s * _NB
    row0 = s * _RPT
    zero16 = jnp.zeros((16,), jnp.float32)

    def chunk_body(kk, cc):
        k = c * 2 + kk
        koff = (k * _NPAD).astype(jnp.int32)

        def zr(i, c2):
            for j8 in range(8):
                r0b[i, pl.ds(j8 * 16, 16)] = zero16
            return c2

        lax.fori_loop(0, _EB, zr, 0)
        for m in range(_RPT // _EB):
            pltpu.async_copy(r0b, acc.at[pl.ds(row0 + m * _EB, _EB)],
                             sg[m % 4])
        for m in range(_RPT // _EB):
            pltpu.make_async_copy(r0b, acc.at[pl.ds(row0, _EB)],
                                  sg[m % 4]).wait()
        plsc.subcore_barrier()

        def g_start(j, b):
            pltpu.async_copy(ps.at[srck.at[j]], rows[b], sg[b])

        def g_wait(b):
            pltpu.make_async_copy(ps.at[pl.ds(0, _EB)], rows[b],
                                  sg[b]).wait()

        def s_start(j, b):
            pltpu.async_copy(rows[b], acc.at[dstt.at[j]], ss[b], add=True)

        def s_wait(b):
            pltpu.make_async_copy(ps.at[pl.ds(0, _EB)], rows[b],
                                  ss[b]).wait()

        def issue_se(seg):
            sb = rb + seg * _SB
            pltpu.async_copy(src2.at[pl.ds(sb, _SB)], srck, sg[0])
            pltpu.async_copy(ew2.at[pl.ds(sb, _SB)], ewt, sg[2])

        def issue_d(seg):
            sb = rb + seg * _SB
            pltpu.async_copy(dst2.at[pl.ds(sb, _SB)], dstt, sg[1])

        issue_se(0)
        issue_d(0)

        for seg in range(_NSEG):
            pltpu.make_async_copy(src2.at[pl.ds(rb, _SB)], srck,
                                  sg[0]).wait()
            pltpu.make_async_copy(dst2.at[pl.ds(rb, _SB)], dstt,
                                  sg[1]).wait()
            pltpu.make_async_copy(ew2.at[pl.ds(rb, _SB)], ewt,
                                  sg[2]).wait()

            kv = jnp.full((16,), 0, jnp.int32) + koff

            def addk(i, c2):
                for g in range(_EB // 16):
                    srck[i, pl.ds(g * 16, 16)] = (
                        srck[i, pl.ds(g * 16, 16)] + kv)
                return c2

            lax.fori_loop(0, _SB, addk, 0)

            g_start(0, 0)
            g_start(1, 1)

            def ring_iter(it, c2):
                for b in range(4):
                    j = it * 4 + b
                    nb = (b + 2) % 4

                    @pl.when((j >= 2) & (j + 2 < _SB))
                    def _sw():
                        s_wait(nb)

                    @pl.when(j + 2 < _SB)
                    def _gs():
                        g_start(j + 2, nb)

                    g_wait(b)

                    def grp(g, c3):
                        base = pl.multiple_of(g * 16, 16)
                        wvec = ewt[j, pl.ds(base, 16)]
                        buf = rows[b]
                        for i16 in range(16):
                            wv = jnp.broadcast_to(wvec[i16:i16 + 1], (16,))
                            r = base + i16
                            for j8 in range(8):
                                buf[r, pl.ds(j8 * 16, 16)] = (
                                    buf[r, pl.ds(j8 * 16, 16)] * wv)
                        return c3

                    lax.fori_loop(0, _EB // 16, grp, 0)
                    s_start(j, b)
                return c2

            lax.fori_loop(0, _SB // 4, ring_iter, 0)
            if seg + 1 < _NSEG:
                issue_se(seg + 1)
            for b in range(4):
                s_wait(b)
            if seg + 1 < _NSEG:
                issue_d(seg + 1)

        plsc.subcore_barrier()
        wb = (r0b, r1b)
        nwb = _RPT // _EB
        pltpu.async_copy(acc.at[pl.ds(row0, _EB)], r0b, sg[0])
        for m in range(nwb):
            bm = wb[m % 2]
            pltpu.make_async_copy(acc.at[pl.ds(row0, _EB)], bm,
                                  sg[m % 2]).wait()
            if m >= 1:
                pltpu.make_async_copy(wb[(m - 1) % 2],
                                      outs.at[pl.ds(koff + row0, _EB)],
                                      ss[(m - 1) % 2]).wait()
            if m + 1 < nwb:
                pltpu.async_copy(acc.at[pl.ds(row0 + (m + 1) * _EB, _EB)],
                                 wb[(m + 1) % 2], sg[(m + 1) % 2])
            pltpu.async_copy(bm, outs.at[pl.ds(koff + row0 + m * _EB, _EB)],
                             ss[m % 2])
        pltpu.make_async_copy(wb[(nwb - 1) % 2],
                              outs.at[pl.ds(koff + row0, _EB)],
                              ss[(nwb - 1) % 2]).wait()
        return cc

    lax.fori_loop(0, _NPLANE // 2, chunk_body, 0)


_seg = pl.kernel(
    _seg_body,
    out_type=jax.ShapeDtypeStruct((_NPLANE * _NPAD, _FC), jnp.float32),
    mesh=plsc.VectorSubcoreMesh(core_axis_name="c", subcore_axis_name="s"),
    scratch_types=[
        pltpu.VMEM((_SB, _EB), jnp.int32),
        pltpu.VMEM((_SB, _EB), jnp.int32),
        pltpu.VMEM((_SB, _EB), jnp.float32),
        pltpu.VMEM((_EB, _FC), jnp.float32),
        pltpu.VMEM((_EB, _FC), jnp.float32),
        pltpu.VMEM((_EB, _FC), jnp.float32),
        pltpu.VMEM((_EB, _FC), jnp.float32),
        pltpu.SemaphoreType.DMA,
        pltpu.SemaphoreType.DMA,
        pltpu.SemaphoreType.DMA,
        pltpu.SemaphoreType.DMA,
        pltpu.SemaphoreType.DMA,
        pltpu.SemaphoreType.DMA,
        pltpu.SemaphoreType.DMA,
        pltpu.SemaphoreType.DMA,
        pltpu.VMEM_SHARED((_NPAD, _FC), jnp.float32),
    ],
)


def _leaky(x):
    return jnp.where(x >= 0, x, 0.01 * x)


def _mm_in_body(x_ref, w_ref, o_ref):
    y = jnp.dot(x_ref[...], w_ref[...], preferred_element_type=jnp.float32)
    for kk in range(_NPLANE):
        o_ref[kk] = y[:, kk * _FC:(kk + 1) * _FC]


def _mm_mid_body(a_ref, b_ref, w_ref, o_ref):
    h = jnp.concatenate([a_ref[kk] for kk in range(_NPLANE)], axis=1)
    h = _leaky(h + b_ref[...])
    y = jnp.dot(h, w_ref[...], preferred_element_type=jnp.float32)
    for kk in range(_NPLANE):
        o_ref[kk] = y[:, kk * _FC:(kk + 1) * _FC]


def _mm_out_body(a_ref, b_ref, w_ref, bl_ref, o_ref):
    h = jnp.concatenate([a_ref[kk] for kk in range(_NPLANE)], axis=1)
    h = _leaky(h + b_ref[...])
    o_ref[...] = (jnp.dot(h, w_ref[...], preferred_element_type=jnp.float32)
                  + bl_ref[...])


_STACK_SPEC = pl.BlockSpec((_NPLANE, _BM, _FC), lambda i: (0, i, 0))


def _mm_in(x, w):
    kin = x.shape[1]
    return pl.pallas_call(
        _mm_in_body,
        grid=(_N // _BM,),
        in_specs=[
            pl.BlockSpec((_BM, kin), lambda i: (i, 0)),
            pl.BlockSpec((kin, w.shape[1]), lambda i: (0, 0)),
        ],
        out_specs=_STACK_SPEC,
        out_shape=jax.ShapeDtypeStruct((_NPLANE, _NPAD, _FC), jnp.float32),
    )(x, w)


def _mm_mid(a, b, w):
    return pl.pallas_call(
        _mm_mid_body,
        grid=(_N // _BM,),
        in_specs=[
            _STACK_SPEC,
            pl.BlockSpec((1, b.shape[1]), lambda i: (0, 0)),
            pl.BlockSpec((w.shape[0], w.shape[1]), lambda i: (0, 0)),
        ],
        out_specs=_STACK_SPEC,
        out_shape=jax.ShapeDtypeStruct((_NPLANE, _NPAD, _FC), jnp.float32),
    )(a, b, w)


def _mm_out(a, b, w, bl):
    ncls = w.shape[1]
    return pl.pallas_call(
        _mm_out_body,
        grid=(_N // _BM,),
        in_specs=[
            _STACK_SPEC,
            pl.BlockSpec((1, b.shape[1]), lambda i: (0, 0)),
            pl.BlockSpec((w.shape[0], ncls), lambda i: (0, 0)),
            pl.BlockSpec((1, ncls), lambda i: (0, 0)),
        ],
        out_specs=pl.BlockSpec((_BM, ncls), lambda i: (i, 0)),
        out_shape=jax.ShapeDtypeStruct((_N, ncls), jnp.float32),
    )(a, b, w, bl)


def kernel(x, edge_index, aw0, aw1, W1, b1, W2, b2, Wl, bl):
    npad = _EROWS * _EB - _E
    src2 = jnp.pad(edge_index[0], (0, npad)).reshape(_EROWS, _EB)
    dst2 = jnp.pad(edge_index[1], (0, npad)).reshape(_EROWS, _EB)
    ew0 = jnp.pad(aw0, (0, npad)).reshape(_EROWS, _EB)
    ew1 = jnp.pad(aw1, (0, npad)).reshape(_EROWS, _EB)
    flat = (_NPLANE * _NPAD, _FC)
    stck = (_NPLANE, _NPAD, _FC)
    p = _mm_in(x, W1)
    a = _seg(p.reshape(flat), src2, dst2, ew0)
    q = _mm_mid(a.reshape(stck), b1.reshape(1, -1), W2)
    g = _seg(q.reshape(flat), src2, dst2, ew1)
    return _mm_out(g.reshape(stck), b2.reshape(1, -1), Wl, bl.reshape(1, -1))
